# baseline (speedup 1.0000x reference)
import functools
import jax
import jax.numpy as jnp
from jax import lax
from jax.experimental import pallas as pl
from jax.experimental.pallas import tpu as pltpu
from jax.experimental.pallas import tpu_sc as plsc

_N = 20000
_C = 128
_G = 1024
_H = 4
_HD = _C // _H
_ROWS = 160
_NPAD = _ROWS * 128
_BIGI = (1 << 30)


def _fps_body(x_ref, y_ref, z_ref, pk_ref, gidx_ref, cpk_ref, dist_ref):
    flat = (lax.broadcasted_iota(jnp.int32, (_ROWS, 128), 0) * 128
            + lax.broadcasted_iota(jnp.int32, (_ROWS, 128), 1))
    rcomp = (_ROWS - 1) - lax.broadcasted_iota(jnp.int32, (_ROWS, 128), 0)
    pkl = (pk_ref[:]
           | ((127 - lax.broadcasted_iota(jnp.int32, (_ROWS, 128), 1)) << 24))
    dist_ref[:] = jnp.where(flat < _N, jnp.float32(1e10), jnp.float32(-1.0))
    x = x_ref[:]
    y = y_ref[:]
    z = z_ref[:]
    oi = (lax.broadcasted_iota(jnp.int32, (8, 128), 0) * 128
          + lax.broadcasted_iota(jnp.int32, (8, 128), 1))

    def fold2(k, p, h):
        t = k[:h] >= k[h:]
        return jnp.where(t, k[:h], k[h:]), jnp.where(t, p[:h], p[h:])

    def body(i, carry):
        far, cpk_s = carry
        gidx_ref[:] = jnp.where(oi == i, far, gidx_ref[:])
        cpk_ref[:] = jnp.where(oi == i, cpk_s, cpk_ref[:])
        pkb = jnp.broadcast_to(cpk_s, (_ROWS, 128))
        cx = (pkb & 255).astype(jnp.float32)
        cy = ((pkb >> 8) & 255).astype(jnp.float32)
        cz = ((pkb >> 16) & 255).astype(jnp.float32)
        dx = x - cx
        dy = y - cy
        dz = z - cz
        d = dx * dx + dy * dy + dz * dz
        nd = jnp.minimum(dist_ref[:], d)
        dist_ref[:] = nd
        ki = nd.astype(jnp.int32) * 256 + rcomp
        k, p = fold2(ki, pkl, 80)
        k, p = fold2(k, p, 40)
        ka, pa = k[:16], p[:16]
        kb, pb = k[16:32], p[16:32]
        kc, pc = k[32:40], p[32:40]
        t = ka >= kb
        k, p = jnp.where(t, ka, kb), jnp.where(t, pa, pb)
        k, p = fold2(k, p, 8)
        t = k >= kc
        k, p = jnp.where(t, k, kc), jnp.where(t, p, pc)
        k1 = jnp.max(jnp.max(k, axis=0, keepdims=True), axis=1, keepdims=True)
        sel = jnp.where(k == k1, p, -1)
        k2 = jnp.max(jnp.max(sel, axis=0, keepdims=True), axis=1, keepdims=True)
        mi = ((_ROWS - 1) - (k1 & 255)) * 128 + (127 - (k2 >> 24))
        return (mi, k2 & 0xFFFFFF)

    far0 = jnp.zeros((1, 1), jnp.int32)
    lax.fori_loop(0, _G, body, (far0, pk_ref[0:1, 0:1]))


_fps = pl.pallas_call(
    _fps_body,
    out_shape=(jax.ShapeDtypeStruct((8, 128), jnp.int32),
               jax.ShapeDtypeStruct((8, 128), jnp.int32)),
    scratch_shapes=[pltpu.VMEM((_ROWS, 128), jnp.float32)],
)


_SC_NC = 2
_SC_NS = 16
_SC_NW = _SC_NC * _SC_NS
_B1W = _G // _SC_NW

def _sc_gather_body(table_hbm, idx_hbm, out_hbm, idx_v, rows_v, sem):
    wid = lax.axis_index("s") * _SC_NC + lax.axis_index("c")
    base = wid * _B1W
    pltpu.sync_copy(idx_hbm.at[pl.ds(base, _B1W)], idx_v)
    pltpu.async_copy(table_hbm.at[idx_v], rows_v, sem).wait()
    pltpu.sync_copy(rows_v, out_hbm.at[pl.ds(base, _B1W)])


@functools.cache
def _get_sc_gather():
    mesh = plsc.VectorSubcoreMesh(
        core_axis_name="c", subcore_axis_name="s",
        num_cores=_SC_NC, num_subcores=_SC_NS)
    return pl.kernel(
        _sc_gather_body, mesh=mesh,
        out_type=jax.ShapeDtypeStruct((_G, _C), jnp.float32),
        scratch_types=[pltpu.VMEM((_B1W,), jnp.int32),
                       pltpu.VMEM((_B1W, _C), jnp.float32),
                       pltpu.SemaphoreType.DMA])


def _sc_gather(table, idx):
    return _get_sc_gather()(table, idx)


def _attn_body(x_ref, wqkv_ref, bqkv_ref, wproj_ref, bproj_ref, o_ref):
    x = x_ref[:]
    qkv = lax.dot_general(x, wqkv_ref[:], (((1,), (0,)), ((), ())),
                          precision=lax.Precision.HIGHEST,
                          preferred_element_type=jnp.float32)
    qkv = qkv + bqkv_ref[0:1, :]
    scale = jnp.float32(_HD ** -0.5)
    outs = []
    for h in range(_H):
        q = qkv[:, h * _HD:(h + 1) * _HD] * scale
        k = qkv[:, _C + h * _HD:_C + (h + 1) * _HD]
        v = qkv[:, 2 * _C + h * _HD:2 * _C + (h + 1) * _HD]
        logits = lax.dot_general(q, k, (((1,), (1,)), ((), ())),
                                 precision=lax.Precision.HIGHEST,
                                 preferred_element_type=jnp.float32)
        mx = jnp.max(logits, axis=1, keepdims=True)
        p = jnp.exp(logits - mx)
        p = p / jnp.sum(p, axis=1, keepdims=True)
        outs.append(lax.dot_general(p, v, (((1,), (0,)), ((), ())),
                                    precision=lax.Precision.HIGHEST,
                                    preferred_element_type=jnp.float32))
    o = jnp.concatenate(outs, axis=1)
    o_ref[:] = (lax.dot_general(o, wproj_ref[:], (((1,), (0,)), ((), ())),
                                precision=lax.Precision.HIGHEST,
                                preferred_element_type=jnp.float32)
                + bproj_ref[0:1, :])


_attn = pl.pallas_call(
    _attn_body,
    out_shape=jax.ShapeDtypeStruct((_G, _C), jnp.float32),
)


_PB = 512
_NBLK = _NPAD // _PB


def _knn_body(p8_ref, c8_ref, gf_ref, feat_ref, out_ref):
    riota = lax.broadcasted_iota(jnp.int32, (_G, 128), 0)
    riotab = lax.broadcasted_iota(jnp.int32, (_G, _PB), 0)
    dmat = lax.dot_general(c8_ref[:], p8_ref[0], (((1,), (0,)), ((), ())),
                           precision=lax.Precision.HIGHEST,
                           preferred_element_type=jnp.float32)
    key = dmat.astype(jnp.int32) * 1024 + riotab
    kmin = jnp.min(key, axis=0, keepdims=True)
    idx = kmin & 1023
    gf = gf_ref[:]
    for j in range(_PB // 128):
        sl = slice(j * 128, (j + 1) * 128)
        oh = jnp.where(riota == idx[0:1, sl], jnp.float32(1.0),
                       jnp.float32(0.0))
        ctx = lax.dot_general(oh, gf, (((0,), (0,)), ((), ())),
                              precision=lax.Precision.HIGHEST,
                              preferred_element_type=jnp.float32)
        out_ref[sl, :] = ctx + feat_ref[sl, :]


_knn = pl.pallas_call(
    _knn_body,
    grid=(_NBLK,),
    in_specs=[
        pl.BlockSpec((1, 8, _PB), lambda i: (i, 0, 0)),
        pl.BlockSpec((_G, 8), lambda i: (0, 0)),
        pl.BlockSpec((_G, _C), lambda i: (0, 0)),
        pl.BlockSpec((_PB, _C), lambda i: (i, 0)),
    ],
    out_specs=pl.BlockSpec((_PB, _C), lambda i: (i, 0)),
    out_shape=jax.ShapeDtypeStruct((_NPAD, _C), jnp.float32),
)


def _pad_plane(col):
    return jnp.pad(col, (0, _NPAD - _N)).reshape(_ROWS, 128)


def kernel(features, indices, Wqkv, bqkv, Wproj, bproj):
    ix = indices[:, 1]
    iy = indices[:, 2]
    iz = indices[:, 3]
    x2d = _pad_plane(ix.astype(jnp.float32))
    y2d = _pad_plane(iy.astype(jnp.float32))
    z2d = _pad_plane(iz.astype(jnp.float32))
    pk2d = _pad_plane(ix + (iy << 8) + (iz << 16))

    gidx2d, cpk2d = _fps(x2d, y2d, z2d, pk2d)
    group_idx = gidx2d.reshape(_G)

    gathered = _sc_gather(features, group_idx)

    bqkv8 = jnp.broadcast_to(bqkv[None, :], (8, 3 * _C))
    bproj8 = jnp.broadcast_to(bproj[None, :], (8, _C))
    group_features = _attn(gathered, Wqkv, bqkv8, Wproj, bproj8)

    cpk = cpk2d.reshape(_G)
    cx = (cpk & 255).astype(jnp.float32)
    cy = ((cpk >> 8) & 255).astype(jnp.float32)
    cz = ((cpk >> 16) & 255).astype(jnp.float32)
    ones_c = jnp.ones((_G,), jnp.float32)
    c8 = jnp.stack([-2.0 * cx, -2.0 * cy, -2.0 * cz,
                    cx * cx + cy * cy + cz * cz, ones_c,
                    jnp.zeros((_G,), jnp.float32),
                    jnp.zeros((_G,), jnp.float32),
                    jnp.zeros((_G,), jnp.float32)], axis=1)

    xf = x2d.reshape(_NPAD)
    yf = y2d.reshape(_NPAD)
    zf = z2d.reshape(_NPAD)
    p8full = jnp.stack([xf, yf, zf, jnp.ones((_NPAD,), jnp.float32),
                        xf * xf + yf * yf + zf * zf,
                        jnp.zeros((_NPAD,), jnp.float32),
                        jnp.zeros((_NPAD,), jnp.float32),
                        jnp.zeros((_NPAD,), jnp.float32)], axis=0)
    p8 = jnp.swapaxes(p8full.reshape(8, _NBLK, _PB), 0, 1)
    featpad = jnp.pad(features, ((0, _NPAD - _N), (0, 0)))

    out = _knn(p8, c8, group_features, featpad)
    return out[:_N]


# speedup vs baseline: 1.1947x; 1.1947x over previous
import functools
import jax
import jax.numpy as jnp
from jax import lax
from jax.experimental import pallas as pl
from jax.experimental.pallas import tpu as pltpu
from jax.experimental.pallas import tpu_sc as plsc

_N = 20000
_C = 128
_G = 1024
_H = 4
_HD = _C // _H
_ROWS = 160
_NPAD = _ROWS * 128
_BIGI = (1 << 30)


def _fps_body(x_ref, y_ref, z_ref, pk_ref, gidx_ref, cpk_ref, dist_ref):
    flat = (lax.broadcasted_iota(jnp.int32, (_ROWS, 128), 0) * 128
            + lax.broadcasted_iota(jnp.int32, (_ROWS, 128), 1))
    rcomp = (_ROWS - 1) - lax.broadcasted_iota(jnp.int32, (_ROWS, 128), 0)
    pkl = (pk_ref[:]
           | ((127 - lax.broadcasted_iota(jnp.int32, (_ROWS, 128), 1)) << 24))
    dist_ref[:] = jnp.where(flat < _N, jnp.float32(1e10), jnp.float32(-1.0))
    x = x_ref[:]
    y = y_ref[:]
    z = z_ref[:]
    oi = (lax.broadcasted_iota(jnp.int32, (8, 128), 0) * 128
          + lax.broadcasted_iota(jnp.int32, (8, 128), 1))

    def fold2(k, p, h):
        t = k[:h] >= k[h:]
        return jnp.where(t, k[:h], k[h:]), jnp.where(t, p[:h], p[h:])

    def body(i, carry):
        far, cpk_s = carry
        gidx_ref[:] = jnp.where(oi == i, far, gidx_ref[:])
        cpk_ref[:] = jnp.where(oi == i, cpk_s, cpk_ref[:])
        pkb = jnp.broadcast_to(cpk_s, (_ROWS, 128))
        cx = (pkb & 255).astype(jnp.float32)
        cy = ((pkb >> 8) & 255).astype(jnp.float32)
        cz = ((pkb >> 16) & 255).astype(jnp.float32)
        dx = x - cx
        dy = y - cy
        dz = z - cz
        d = dx * dx + dy * dy + dz * dz
        nd = jnp.minimum(dist_ref[:], d)
        dist_ref[:] = nd
        ki = nd.astype(jnp.int32) * 256 + rcomp
        k, p = fold2(ki, pkl, 80)
        k, p = fold2(k, p, 40)
        ka, pa = k[:16], p[:16]
        kb, pb = k[16:32], p[16:32]
        kc, pc = k[32:40], p[32:40]
        t = ka >= kb
        k, p = jnp.where(t, ka, kb), jnp.where(t, pa, pb)
        k, p = fold2(k, p, 8)
        t = k >= kc
        k, p = jnp.where(t, k, kc), jnp.where(t, p, pc)
        k1 = jnp.max(jnp.max(k, axis=0, keepdims=True), axis=1, keepdims=True)
        sel = jnp.where(k == k1, p, -1)
        k2 = jnp.max(jnp.max(sel, axis=0, keepdims=True), axis=1, keepdims=True)
        mi = ((_ROWS - 1) - (k1 & 255)) * 128 + (127 - (k2 >> 24))
        return (mi, k2 & 0xFFFFFF)

    far0 = jnp.zeros((1, 1), jnp.int32)
    lax.fori_loop(0, _G, body, (far0, pk_ref[0:1, 0:1]))


_fps = pl.pallas_call(
    _fps_body,
    out_shape=(jax.ShapeDtypeStruct((8, 128), jnp.int32),
               jax.ShapeDtypeStruct((8, 128), jnp.int32)),
    scratch_shapes=[pltpu.VMEM((_ROWS, 128), jnp.float32)],
)


_SC_NC = 2
_SC_NS = 16
_SC_NW = _SC_NC * _SC_NS
_B1W = _G // _SC_NW

def _sc_gather_body(table_hbm, idx_hbm, out_hbm, idx_v, rows_v, sem):
    wid = lax.axis_index("s") * _SC_NC + lax.axis_index("c")
    base = wid * _B1W
    pltpu.sync_copy(idx_hbm.at[pl.ds(base, _B1W)], idx_v)
    pltpu.async_copy(table_hbm.at[idx_v], rows_v, sem).wait()
    pltpu.sync_copy(rows_v, out_hbm.at[pl.ds(base, _B1W)])


@functools.cache
def _get_sc_gather():
    mesh = plsc.VectorSubcoreMesh(
        core_axis_name="c", subcore_axis_name="s",
        num_cores=_SC_NC, num_subcores=_SC_NS)
    return pl.kernel(
        _sc_gather_body, mesh=mesh,
        out_type=jax.ShapeDtypeStruct((_G, _C), jnp.float32),
        scratch_types=[pltpu.VMEM((_B1W,), jnp.int32),
                       pltpu.VMEM((_B1W, _C), jnp.float32),
                       pltpu.SemaphoreType.DMA])


def _sc_gather(table, idx):
    return _get_sc_gather()(table, idx)


def _attn_body(x_ref, wqkv_ref, bqkv_ref, wproj_ref, bproj_ref, o_ref):
    x = x_ref[:].astype(jnp.bfloat16)
    qkv = lax.dot_general(x, wqkv_ref[:].astype(jnp.bfloat16),
                          (((1,), (0,)), ((), ())),
                          preferred_element_type=jnp.float32)
    qkv = qkv + bqkv_ref[0:1, :]
    scale = jnp.float32(_HD ** -0.5)
    outs = []
    for h in range(_H):
        q = (qkv[:, h * _HD:(h + 1) * _HD] * scale).astype(jnp.bfloat16)
        k = qkv[:, _C + h * _HD:_C + (h + 1) * _HD].astype(jnp.bfloat16)
        v = qkv[:, 2 * _C + h * _HD:2 * _C + (h + 1) * _HD].astype(jnp.bfloat16)
        logits = lax.dot_general(q, k, (((1,), (1,)), ((), ())),
                                 preferred_element_type=jnp.float32)
        mx = jnp.max(logits, axis=1, keepdims=True)
        p = jnp.exp(logits - mx)
        p = (p / jnp.sum(p, axis=1, keepdims=True)).astype(jnp.bfloat16)
        outs.append(lax.dot_general(p, v, (((1,), (0,)), ((), ())),
                                    preferred_element_type=jnp.float32))
    o = jnp.concatenate(outs, axis=1).astype(jnp.bfloat16)
    o_ref[:] = (lax.dot_general(o, wproj_ref[:].astype(jnp.bfloat16),
                                (((1,), (0,)), ((), ())),
                                preferred_element_type=jnp.float32)
                + bproj_ref[0:1, :])


_attn = pl.pallas_call(
    _attn_body,
    out_shape=jax.ShapeDtypeStruct((_G, _C), jnp.float32),
)


_PB = 512
_NBLK = _NPAD // _PB


def _knn_body(px_ref, py_ref, pz_ref, cx_ref, cy_ref, cz_ref, gf_ref,
              feat_ref, out_ref):
    riota = lax.broadcasted_iota(jnp.int32, (_G, 128), 0)
    cx = cx_ref[:]
    cy = cy_ref[:]
    cz = cz_ref[:]
    gfb = gf_ref[:].astype(jnp.bfloat16)
    for j in range(_PB // 128):
        sl = slice(j * 128, (j + 1) * 128)
        dx = px_ref[0, 0:1, sl] - cx
        dy = py_ref[0, 0:1, sl] - cy
        dz = pz_ref[0, 0:1, sl] - cz
        d = dx * dx + dy * dy + dz * dz
        key = d.astype(jnp.int32) * 1024 + riota
        kmin = jnp.min(key, axis=0, keepdims=True)
        idx = kmin & 1023
        oh = jnp.where(riota == idx, jnp.float32(1.0),
                       jnp.float32(0.0)).astype(jnp.bfloat16)
        ctx = lax.dot_general(oh, gfb, (((0,), (0,)), ((), ())),
                              preferred_element_type=jnp.float32)
        out_ref[sl, :] = ctx + feat_ref[sl, :]


_knn = pl.pallas_call(
    _knn_body,
    grid=(_NBLK,),
    in_specs=[
        pl.BlockSpec((1, 1, _PB), lambda i: (i, 0, 0)),
        pl.BlockSpec((1, 1, _PB), lambda i: (i, 0, 0)),
        pl.BlockSpec((1, 1, _PB), lambda i: (i, 0, 0)),
        pl.BlockSpec((_G, 128), lambda i: (0, 0)),
        pl.BlockSpec((_G, 128), lambda i: (0, 0)),
        pl.BlockSpec((_G, 128), lambda i: (0, 0)),
        pl.BlockSpec((_G, _C), lambda i: (0, 0)),
        pl.BlockSpec((_PB, _C), lambda i: (i, 0)),
    ],
    out_specs=pl.BlockSpec((_PB, _C), lambda i: (i, 0)),
    out_shape=jax.ShapeDtypeStruct((_NPAD, _C), jnp.float32),
)


def _pad_plane(col):
    return jnp.pad(col, (0, _NPAD - _N)).reshape(_ROWS, 128)


def kernel(features, indices, Wqkv, bqkv, Wproj, bproj):
    ix = indices[:, 1]
    iy = indices[:, 2]
    iz = indices[:, 3]
    x2d = _pad_plane(ix.astype(jnp.float32))
    y2d = _pad_plane(iy.astype(jnp.float32))
    z2d = _pad_plane(iz.astype(jnp.float32))
    pk2d = _pad_plane(ix + (iy << 8) + (iz << 16))

    gidx2d, cpk2d = _fps(x2d, y2d, z2d, pk2d)
    group_idx = gidx2d.reshape(_G)

    gathered = _sc_gather(features, group_idx)

    bqkv8 = jnp.broadcast_to(bqkv[None, :], (8, 3 * _C))
    bproj8 = jnp.broadcast_to(bproj[None, :], (8, _C))
    group_features = _attn(gathered, Wqkv, bqkv8, Wproj, bproj8)

    cpk = cpk2d.reshape(_G)
    cxb = jnp.broadcast_to((cpk & 255).astype(jnp.float32)[:, None], (_G, 128))
    cyb = jnp.broadcast_to(((cpk >> 8) & 255).astype(jnp.float32)[:, None],
                           (_G, 128))
    czb = jnp.broadcast_to(((cpk >> 16) & 255).astype(jnp.float32)[:, None],
                           (_G, 128))
    px = x2d.reshape(_NBLK, 1, _PB)
    py = y2d.reshape(_NBLK, 1, _PB)
    pz = z2d.reshape(_NBLK, 1, _PB)
    featpad = jnp.pad(features, ((0, _NPAD - _N), (0, 0)))

    out = _knn(px, py, pz, cxb, cyb, czb, group_features, featpad)
    return out[:_N]
